# baseline (speedup 1.0000x reference)
import functools

import jax
import jax.numpy as jnp
from jax import lax
from jax.experimental import pallas as pl
from jax.experimental.pallas import tpu as pltpu
from jax.experimental.pallas import tpu_sc as plsc

N_TOK = 32768
D = 4096
E = 64
TOPK = 8
BT = 512
NC, NS = 2, 16
NW = NC * NS
CHUNKS = (8192, 8192, 8192, 4096, 2048, 2048)

_GATHER_DNUMS = lax.GatherDimensionNumbers(
    offset_dims=(), collapsed_slice_dims=(0,), start_index_map=(0,))


def _take16(a, idx):
    return lax.gather(a, idx[:, None], _GATHER_DNUMS, slice_sizes=(1,),
                      mode=lax.GatherScatterMode.PROMISE_IN_BOUNDS)


def _matmul_block(x_ref, wt_ref, logits_ref):
    logits_ref[...] = jnp.dot(x_ref[...], wt_ref[...],
                              preferred_element_type=jnp.float32)


def _sc_router_body(tpw, logits_hbm, mask_hbm, scores_hbm, idx_hbm, sums_hbm,
                    logits_v, scores_v, mask_v, idx_v, sums_v, shared, all_v):
    cid = lax.axis_index("c")
    sid = lax.axis_index("s")
    wid = sid * NC + cid
    tok0 = wid * tpw

    lane = lax.iota(jnp.int32, 16)
    half = lane < 8
    shifted = jnp.where(half, lane, lane - 8)
    ones16 = jnp.full((16,), 1.0, jnp.float32)
    zeros16 = jnp.zeros((16,), jnp.float32)
    lane15 = jnp.full((16,), 15, jnp.int32)

    for g in range(8):
        sums_v[pl.ds(g * 16, 16)] = zeros16

    def merge(ak, av, bk, bv):
        k = jnp.where(half, _take16(ak, shifted), _take16(bk, shifted))
        v = jnp.where(half, _take16(av, shifted), _take16(bv, shifted))
        return plsc.sort_key_val(k, v, descending=True)

    def top8(sc):
        srt = [plsc.sort_key_val(sc[g], lane + 16 * g, descending=True)
               for g in range(4)]
        k01, v01 = merge(srt[0][0], srt[0][1], srt[1][0], srt[1][1])
        k23, v23 = merge(srt[2][0], srt[2][1], srt[3][0], srt[3][1])
        _, vf = merge(k01, v01, k23, v23)
        return vf

    def token(tok):
        s = [logits_v[tok, pl.ds(16 * g, 16)] for g in range(4)]
        m4 = jnp.maximum(jnp.maximum(s[0], s[1]), jnp.maximum(s[2], s[3]))
        mx = _take16(plsc.cummax(m4), lane15)
        e = [jnp.exp(sg - mx) for sg in s]
        ssum = _take16(plsc.cumsum((e[0] + e[1]) + (e[2] + e[3])), lane15)
        sc = [eg / ssum for eg in e]
        for g in range(4):
            scores_v[tok, pl.ds(16 * g, 16)] = sc[g]
            plsc.addupdate(sums_v.at[pl.ds(g * 16, 16)], sc[g])
            mask_v[tok, pl.ds(16 * g, 16)] = zeros16
        return top8(sc)

    pltpu.sync_copy(logits_hbm.at[pl.ds(tok0, tpw)], logits_v)

    def pair(j, carry):
        tok_a = j * 2
        tok_b = tok_a + 1
        va = token(tok_a)
        vb = token(tok_b)
        iv = jnp.where(half, va, _take16(vb, shifted))
        idx_v[pl.ds(j * 16, 16)] = iv
        row = jnp.where(half, tok_a, tok_b)
        plsc.store_scatter(mask_v, [row, iv], ones16)
        for g in range(4):
            ma = mask_v[tok_a, pl.ds(16 * g, 16)]
            mb = mask_v[tok_b, pl.ds(16 * g, 16)]
            plsc.addupdate(sums_v.at[pl.ds(64 + g * 16, 16)], ma + mb)
        return carry

    lax.fori_loop(0, tpw // 2, pair, 0)

    pltpu.sync_copy(scores_v, scores_hbm.at[pl.ds(tok0, tpw)])
    pltpu.sync_copy(mask_v, mask_hbm.at[pl.ds(tok0, tpw)])
    pltpu.sync_copy(idx_v.at[pl.ds(0, tpw * TOPK)],
                    idx_hbm.at[pl.ds(tok0 * TOPK, tpw * TOPK)])

    pltpu.sync_copy(sums_v, shared.at[pl.ds(sid * 128, 128)])
    plsc.subcore_barrier()

    @pl.when(sid == 0)
    def _():
        pltpu.sync_copy(shared, all_v)
        for g in range(8):
            acc = all_v[pl.ds(g * 16, 16)]
            for r in range(1, NS):
                acc = acc + all_v[pl.ds(r * 128 + g * 16, 16)]
            sums_v[pl.ds(g * 16, 16)] = acc
        pltpu.sync_copy(sums_v, sums_hbm.at[cid])


def _assemble_block(*refs):
    nch = len(CHUNKS)
    mask_in = refs[0:2 * nch:2]
    scores_in = refs[1:2 * nch:2]
    sums_ref = refs[2 * nch]
    mask_out, scores_out, aux_out = refs[2 * nch + 1:]
    i = pl.program_id(0)
    start = 0
    for c, ctok in enumerate(CHUNKS):
        s, n = start // BT, ctok // BT
        start += ctok

        @pl.when((i >= s) & (i < s + n))
        def _(c=c):
            mask_out[...] = mask_in[c][...]
            scores_out[...] = scores_in[c][...]

    @pl.when(i == N_TOK // BT - 1)
    def _():
        tot = jnp.sum(sums_ref[...], axis=0, keepdims=True)
        imp = tot[:, 0:E]
        load = tot[:, E:2 * E]
        aux_out[0, 0] = jnp.sum(imp * load) * (E / (N_TOK * N_TOK))


_SC_MESH = plsc.VectorSubcoreMesh(
    core_axis_name="c", subcore_axis_name="s", num_cores=NC, num_subcores=NS)


@functools.cache
def _make_sc_call(ctok):
    tpw = ctok // NW
    return pl.kernel(
        functools.partial(_sc_router_body, tpw),
        out_type=[
            jax.ShapeDtypeStruct((ctok, E), jnp.float32),
            jax.ShapeDtypeStruct((ctok, E), jnp.float32),
            jax.ShapeDtypeStruct((ctok * TOPK,), jnp.int32),
            jax.ShapeDtypeStruct((NC, 128), jnp.float32),
        ],
        mesh=_SC_MESH,
        scratch_types=[
            pltpu.VMEM((tpw, E), jnp.float32),
            pltpu.VMEM((tpw, E), jnp.float32),
            pltpu.VMEM((tpw, E), jnp.float32),
            pltpu.VMEM((((tpw * TOPK + 1023) // 1024) * 1024,), jnp.int32),
            pltpu.VMEM((128,), jnp.float32),
            pltpu.VMEM_SHARED((NS * 128,), jnp.float32),
            pltpu.VMEM((NS * 128,), jnp.float32),
        ],
        compiler_params=pltpu.CompilerParams(needs_layout_passes=False),
    )


@jax.jit
def _router(x, wt):
    masks, scoress, idxs, sumss = [], [], [], []
    base = 0
    for ctok in CHUNKS:
        base_bt = base // BT
        logits = pl.pallas_call(
            _matmul_block,
            grid=(ctok // BT,),
            in_specs=[
                pl.BlockSpec((BT, D), lambda i, b=base_bt: (i + b, 0)),
                pl.BlockSpec((D, E), lambda i: (0, 0)),
            ],
            out_specs=pl.BlockSpec((BT, E), lambda i: (i, 0)),
            out_shape=jax.ShapeDtypeStruct((ctok, E), jnp.float32),
            compiler_params=pltpu.CompilerParams(
                dimension_semantics=("arbitrary",),
            ),
        )(x, wt)
        mask, scores, idx, sums = _make_sc_call(ctok)(logits)
        masks.append(mask)
        scoress.append(scores)
        idxs.append(idx)
        sumss.append(sums)
        base += ctok

    sums_all = jnp.concatenate(sumss, axis=0)

    in_specs = []
    chunk_args = []
    start = 0
    for c, ctok in enumerate(CHUNKS):
        s, n = start // BT, ctok // BT
        start += ctok
        spec = pl.BlockSpec(
            (BT, E), lambda i, s=s, n=n: (jnp.clip(i - s, 0, n - 1), 0))
        in_specs += [spec, spec]
        chunk_args += [masks[c], scoress[c]]
    in_specs.append(pl.BlockSpec((len(CHUNKS) * NC, 128), lambda i: (0, 0)))

    mask, scores, aux = pl.pallas_call(
        _assemble_block,
        grid=(N_TOK // BT,),
        in_specs=in_specs,
        out_specs=[
            pl.BlockSpec((BT, E), lambda i: (i, 0)),
            pl.BlockSpec((BT, E), lambda i: (i, 0)),
            pl.BlockSpec(memory_space=pltpu.SMEM),
        ],
        out_shape=[
            jax.ShapeDtypeStruct((N_TOK, E), jnp.float32),
            jax.ShapeDtypeStruct((N_TOK, E), jnp.float32),
            jax.ShapeDtypeStruct((1, 1), jnp.float32),
        ],
        compiler_params=pltpu.CompilerParams(
            dimension_semantics=("arbitrary",),
        ),
    )(*chunk_args, sums_all)

    return (mask, scores, aux[0, 0],
            jnp.concatenate(idxs, axis=0).reshape(N_TOK, TOPK))


def kernel(x, W):
    return _router(x, W.T)


# speedup vs baseline: 1.1422x; 1.1422x over previous
import functools

import jax
import jax.numpy as jnp
from jax import lax
from jax.experimental import pallas as pl
from jax.experimental.pallas import tpu as pltpu
from jax.experimental.pallas import tpu_sc as plsc

N_TOK = 32768
D = 4096
E = 64
TOPK = 8
BT = 512
NC, NS = 2, 16
NW = NC * NS
CHUNKS = (8192, 8192, 8192, 4096, 2048, 2048)

_GATHER_DNUMS = lax.GatherDimensionNumbers(
    offset_dims=(), collapsed_slice_dims=(0,), start_index_map=(0,))


def _take16(a, idx):
    return lax.gather(a, idx[:, None], _GATHER_DNUMS, slice_sizes=(1,),
                      mode=lax.GatherScatterMode.PROMISE_IN_BOUNDS)


def _matmul_block(x_ref, wt_ref, logits_ref):
    logits_ref[...] = jnp.dot(x_ref[...], wt_ref[...],
                              preferred_element_type=jnp.float32)


def _sc_router_body(tpw, logits_hbm, mask_hbm, scores_hbm, idx_hbm, sums_hbm,
                    logits_v, scores_v, mask_v, idx_v, sums_v, shared, all_v):
    cid = lax.axis_index("c")
    sid = lax.axis_index("s")
    wid = sid * NC + cid
    tok0 = wid * tpw

    lane = lax.iota(jnp.int32, 16)
    half = lane < 8
    shifted = jnp.where(half, lane, lane - 8)
    ones16 = jnp.full((16,), 1.0, jnp.float32)
    zeros16 = jnp.zeros((16,), jnp.float32)
    lane15 = jnp.full((16,), 15, jnp.int32)

    for g in range(8):
        sums_v[pl.ds(g * 16, 16)] = zeros16

    def merge(ak, av, bk, bv):
        k = jnp.where(half, _take16(ak, shifted), _take16(bk, shifted))
        v = jnp.where(half, _take16(av, shifted), _take16(bv, shifted))
        return plsc.sort_key_val(k, v, descending=True)

    def top8(sc):
        srt = [plsc.sort_key_val(sc[g], lane + 16 * g, descending=True)
               for g in range(4)]
        k01, v01 = merge(srt[0][0], srt[0][1], srt[1][0], srt[1][1])
        k23, v23 = merge(srt[2][0], srt[2][1], srt[3][0], srt[3][1])
        _, vf = merge(k01, v01, k23, v23)
        return vf

    def token(tok):
        s = [logits_v[tok, pl.ds(16 * g, 16)] for g in range(4)]
        m4 = jnp.maximum(jnp.maximum(s[0], s[1]), jnp.maximum(s[2], s[3]))
        mx = _take16(plsc.cummax(m4), lane15)
        e = [jnp.exp(sg - mx) for sg in s]
        ssum = _take16(plsc.cumsum((e[0] + e[1]) + (e[2] + e[3])), lane15)
        sc = [eg / ssum for eg in e]
        for g in range(4):
            scores_v[tok, pl.ds(16 * g, 16)] = sc[g]
            plsc.addupdate(sums_v.at[pl.ds(g * 16, 16)], sc[g])
            mask_v[tok, pl.ds(16 * g, 16)] = zeros16
        return top8(sc)

    pltpu.sync_copy(logits_hbm.at[pl.ds(tok0, tpw)], logits_v)

    def pair(j, carry):
        tok_a = j * 2
        tok_b = tok_a + 1
        va = token(tok_a)
        vb = token(tok_b)
        iv = jnp.where(half, va, _take16(vb, shifted))
        idx_v[pl.ds(j * 16, 16)] = iv
        row = jnp.where(half, tok_a, tok_b)
        plsc.store_scatter(mask_v, [row, iv], ones16)
        for g in range(4):
            ma = mask_v[tok_a, pl.ds(16 * g, 16)]
            mb = mask_v[tok_b, pl.ds(16 * g, 16)]
            plsc.addupdate(sums_v.at[pl.ds(64 + g * 16, 16)], ma + mb)
        return carry

    lax.fori_loop(0, tpw // 2, pair, 0)

    pltpu.sync_copy(scores_v, scores_hbm.at[pl.ds(tok0, tpw)])
    pltpu.sync_copy(mask_v, mask_hbm.at[pl.ds(tok0, tpw)])
    pltpu.sync_copy(idx_v.at[pl.ds(0, tpw * TOPK)],
                    idx_hbm.at[pl.ds(tok0 * TOPK, tpw * TOPK)])

    pltpu.sync_copy(sums_v, shared.at[pl.ds(sid * 128, 128)])
    plsc.subcore_barrier()

    @pl.when(sid == 0)
    def _():
        pltpu.sync_copy(shared, all_v)
        for g in range(8):
            acc = all_v[pl.ds(g * 16, 16)]
            for r in range(1, NS):
                acc = acc + all_v[pl.ds(r * 128 + g * 16, 16)]
            sums_v[pl.ds(g * 16, 16)] = acc
        pltpu.sync_copy(sums_v, sums_hbm.at[cid])


def _aux_block(sums_ref, aux_ref):
    tot = jnp.sum(sums_ref[...], axis=0, keepdims=True)
    imp = tot[:, 0:E]
    load = tot[:, E:2 * E]
    aux_ref[0, 0] = jnp.sum(imp * load) * (E / (N_TOK * N_TOK))


_SC_MESH = plsc.VectorSubcoreMesh(
    core_axis_name="c", subcore_axis_name="s", num_cores=NC, num_subcores=NS)


@functools.cache
def _make_sc_call(ctok):
    tpw = ctok // NW
    return pl.kernel(
        functools.partial(_sc_router_body, tpw),
        out_type=[
            jax.ShapeDtypeStruct((ctok, E), jnp.float32),
            jax.ShapeDtypeStruct((ctok, E), jnp.float32),
            jax.ShapeDtypeStruct((ctok * TOPK,), jnp.int32),
            jax.ShapeDtypeStruct((NC, 128), jnp.float32),
        ],
        mesh=_SC_MESH,
        scratch_types=[
            pltpu.VMEM((tpw, E), jnp.float32),
            pltpu.VMEM((tpw, E), jnp.float32),
            pltpu.VMEM((tpw, E), jnp.float32),
            pltpu.VMEM((((tpw * TOPK + 1023) // 1024) * 1024,), jnp.int32),
            pltpu.VMEM((128,), jnp.float32),
            pltpu.VMEM_SHARED((NS * 128,), jnp.float32),
            pltpu.VMEM((NS * 128,), jnp.float32),
        ],
        compiler_params=pltpu.CompilerParams(needs_layout_passes=False),
    )


@jax.jit
def _router(x, wt):
    masks, scoress, idxs, sumss = [], [], [], []
    base = 0
    for ctok in CHUNKS:
        base_bt = base // BT
        logits = pl.pallas_call(
            _matmul_block,
            grid=(ctok // BT,),
            in_specs=[
                pl.BlockSpec((BT, D), lambda i, b=base_bt: (i + b, 0)),
                pl.BlockSpec((D, E), lambda i: (0, 0)),
            ],
            out_specs=pl.BlockSpec((BT, E), lambda i: (i, 0)),
            out_shape=jax.ShapeDtypeStruct((ctok, E), jnp.float32),
            compiler_params=pltpu.CompilerParams(
                dimension_semantics=("arbitrary",),
            ),
        )(x, wt)
        mask, scores, idx, sums = _make_sc_call(ctok)(logits)
        masks.append(mask)
        scoress.append(scores)
        idxs.append(idx)
        sumss.append(sums)
        base += ctok

    sums_all = jnp.concatenate(sumss, axis=0)
    aux = pl.pallas_call(
        _aux_block,
        in_specs=[pl.BlockSpec((len(CHUNKS) * NC, 128), lambda: (0, 0))],
        out_specs=pl.BlockSpec(memory_space=pltpu.SMEM),
        out_shape=jax.ShapeDtypeStruct((1, 1), jnp.float32),
    )(sums_all)

    mask = jnp.zeros((N_TOK, E), jnp.float32)
    scores = jnp.zeros((N_TOK, E), jnp.float32)
    idx = jnp.zeros((N_TOK * TOPK,), jnp.int32)
    start = 0
    for c, ctok in enumerate(CHUNKS):
        mask = lax.dynamic_update_slice(mask, masks[c], (start, 0))
        scores = lax.dynamic_update_slice(scores, scoress[c], (start, 0))
        idx = lax.dynamic_update_slice(idx, idxs[c], (start * TOPK,))
        start += ctok

    return (mask, scores, aux[0, 0], idx.reshape(N_TOK, TOPK))


def kernel(x, W):
    return _router(x, W.T)


# speedup vs baseline: 1.1830x; 1.0357x over previous
import functools

import jax
import jax.numpy as jnp
from jax import lax
from jax.experimental import pallas as pl
from jax.experimental.pallas import tpu as pltpu
from jax.experimental.pallas import tpu_sc as plsc

N_TOK = 32768
D = 4096
E = 64
TOPK = 8
BT = 512
NC, NS = 2, 16
NW = NC * NS
CHUNKS = (8192, 8192, 8192, 4096, 2048, 2048)

_GATHER_DNUMS = lax.GatherDimensionNumbers(
    offset_dims=(), collapsed_slice_dims=(0,), start_index_map=(0,))


def _take16(a, idx):
    return lax.gather(a, idx[:, None], _GATHER_DNUMS, slice_sizes=(1,),
                      mode=lax.GatherScatterMode.PROMISE_IN_BOUNDS)


def _matmul_block(x_ref, wt_ref, logits_ref):
    logits_ref[...] = jnp.dot(x_ref[...], wt_ref[...],
                              preferred_element_type=jnp.float32)


def _sc_router_body(tpw, logits_hbm, mask_hbm, scores_hbm, idx_hbm, sums_hbm,
                    logits_v, scores_v, mask_v, idx_v, sums_v, shared, all_v):
    cid = lax.axis_index("c")
    sid = lax.axis_index("s")
    wid = sid * NC + cid
    tok0 = wid * tpw

    lane = lax.iota(jnp.int32, 16)
    half = lane < 8
    shifted = jnp.where(half, lane, lane - 8)
    ones16 = jnp.full((16,), 1.0, jnp.float32)
    zeros16 = jnp.zeros((16,), jnp.float32)
    lane15 = jnp.full((16,), 15, jnp.int32)

    for g in range(8):
        sums_v[pl.ds(g * 16, 16)] = zeros16

    def merge(ak, av, bk, bv):
        k = jnp.where(half, _take16(ak, shifted), _take16(bk, shifted))
        v = jnp.where(half, _take16(av, shifted), _take16(bv, shifted))
        return plsc.sort_key_val(k, v, descending=True)

    def top8(sc):
        srt = [plsc.sort_key_val(sc[g], lane + 16 * g, descending=True)
               for g in range(4)]
        k01, v01 = merge(srt[0][0], srt[0][1], srt[1][0], srt[1][1])
        k23, v23 = merge(srt[2][0], srt[2][1], srt[3][0], srt[3][1])
        _, vf = merge(k01, v01, k23, v23)
        return vf

    def token(tok):
        s = [logits_v[tok, pl.ds(16 * g, 16)] for g in range(4)]
        m4 = jnp.maximum(jnp.maximum(s[0], s[1]), jnp.maximum(s[2], s[3]))
        mx = _take16(plsc.cummax(m4), lane15)
        e = [jnp.exp(sg - mx) for sg in s]
        ssum = _take16(plsc.cumsum((e[0] + e[1]) + (e[2] + e[3])), lane15)
        sc = [eg / ssum for eg in e]
        for g in range(4):
            scores_v[tok, pl.ds(16 * g, 16)] = sc[g]
            plsc.addupdate(sums_v.at[pl.ds(g * 16, 16)], sc[g])
            mask_v[tok, pl.ds(16 * g, 16)] = zeros16
        return top8(sc)

    pltpu.sync_copy(logits_hbm.at[pl.ds(tok0, tpw)], logits_v)

    def pair(j, carry):
        tok_a = j * 2
        tok_b = tok_a + 1
        va = token(tok_a)
        vb = token(tok_b)
        iv = jnp.where(half, va, _take16(vb, shifted))
        idx_v[pl.ds(j * 16, 16)] = iv
        row = jnp.where(half, tok_a, tok_b)
        plsc.store_scatter(mask_v, [row, iv], ones16)
        for g in range(4):
            ma = mask_v[tok_a, pl.ds(16 * g, 16)]
            mb = mask_v[tok_b, pl.ds(16 * g, 16)]
            plsc.addupdate(sums_v.at[pl.ds(64 + g * 16, 16)], ma + mb)
        return carry

    lax.fori_loop(0, tpw // 2, pair, 0)

    pltpu.sync_copy(scores_v, scores_hbm.at[pl.ds(tok0, tpw)])
    pltpu.sync_copy(mask_v, mask_hbm.at[pl.ds(tok0, tpw)])
    pltpu.sync_copy(idx_v.at[pl.ds(0, tpw * TOPK)],
                    idx_hbm.at[pl.ds(tok0 * TOPK, tpw * TOPK)])

    pltpu.sync_copy(sums_v, shared.at[pl.ds(sid * 128, 128)])
    plsc.subcore_barrier()

    @pl.when(sid == 0)
    def _():
        pltpu.sync_copy(shared, all_v)
        for g in range(8):
            acc = all_v[pl.ds(g * 16, 16)]
            for r in range(1, NS):
                acc = acc + all_v[pl.ds(r * 128 + g * 16, 16)]
            sums_v[pl.ds(g * 16, 16)] = acc
        pltpu.sync_copy(sums_v, sums_hbm.at[cid])


def _aux_block(sums_ref, aux_ref):
    tot = jnp.sum(sums_ref[...], axis=0, keepdims=True)
    imp = tot[:, 0:E]
    load = tot[:, E:2 * E]
    aux_ref[0, 0] = jnp.sum(imp * load) * (E / (N_TOK * N_TOK))


_SC_MESH = plsc.VectorSubcoreMesh(
    core_axis_name="c", subcore_axis_name="s", num_cores=NC, num_subcores=NS)


@functools.cache
def _make_sc_call(ctok):
    tpw = ctok // NW
    return pl.kernel(
        functools.partial(_sc_router_body, tpw),
        out_type=[
            jax.ShapeDtypeStruct((ctok, E), jnp.float32),
            jax.ShapeDtypeStruct((ctok, E), jnp.float32),
            jax.ShapeDtypeStruct((ctok * TOPK,), jnp.int32),
            jax.ShapeDtypeStruct((NC, 128), jnp.float32),
        ],
        mesh=_SC_MESH,
        scratch_types=[
            pltpu.VMEM((tpw, E), jnp.float32),
            pltpu.VMEM((tpw, E), jnp.float32),
            pltpu.VMEM((tpw, E), jnp.float32),
            pltpu.VMEM((((tpw * TOPK + 1023) // 1024) * 1024,), jnp.int32),
            pltpu.VMEM((128,), jnp.float32),
            pltpu.VMEM_SHARED((NS * 128,), jnp.float32),
            pltpu.VMEM((NS * 128,), jnp.float32),
        ],
        compiler_params=pltpu.CompilerParams(needs_layout_passes=False),
    )


@jax.jit
def _router(x, wt):
    masks, scoress, idxs, sumss = [], [], [], []
    base = 0
    for ctok in CHUNKS:
        base_bt = base // BT
        logits = pl.pallas_call(
            _matmul_block,
            grid=(ctok // BT,),
            in_specs=[
                pl.BlockSpec((BT, D), lambda i, b=base_bt: (i + b, 0)),
                pl.BlockSpec((D, E), lambda i: (0, 0)),
            ],
            out_specs=pl.BlockSpec((BT, E), lambda i: (i, 0)),
            out_shape=jax.ShapeDtypeStruct((ctok, E), jnp.float32),
            compiler_params=pltpu.CompilerParams(
                dimension_semantics=("arbitrary",),
            ),
        )(x, wt)
        mask, scores, idx, sums = _make_sc_call(ctok)(logits)
        masks.append(mask)
        scoress.append(scores)
        idxs.append(idx)
        sumss.append(sums)
        base += ctok

    sums_all = jnp.concatenate(sumss, axis=0)
    aux = pl.pallas_call(
        _aux_block,
        in_specs=[pl.BlockSpec((len(CHUNKS) * NC, 128), lambda: (0, 0))],
        out_specs=pl.BlockSpec(memory_space=pltpu.SMEM),
        out_shape=jax.ShapeDtypeStruct((1, 1), jnp.float32),
    )(sums_all)

    return (jnp.concatenate(masks, axis=0),
            jnp.concatenate(scoress, axis=0),
            aux[0, 0],
            jnp.concatenate(idxs, axis=0).reshape(N_TOK, TOPK))


def kernel(x, W):
    return _router(x, W.T)
